# baseline (speedup 1.0000x reference)
import functools

import jax
import jax.numpy as jnp
from jax.experimental import pallas as pl
from jax.experimental.pallas import tpu as pltpu


def _xwt(xf, w):
    return jax.lax.dot_general(xf, w, (((1,), (1,)), ((), ())),
                               preferred_element_type=jnp.float32)


def _gnn_block(out_dim, a_ud_ref, a_lr_ref, xf_ref, wlin_ref, wud_ref,
               wlr_ref, blin_ref, bud_ref, blr_ref, out_ref, y_ref, base_ref):
    i = pl.program_id(0)

    @pl.when(i == 0)
    def _():
        xf = xf_ref[...]
        y_ref[:, :out_dim] = _xwt(xf, wud_ref[...])
        y_ref[:, out_dim:] = _xwt(xf, wlr_ref[...])
        base_ref[...] = (_xwt(xf, wlin_ref[...])
                         + (blin_ref[...] + bud_ref[...] + blr_ref[...]))

    y = y_ref[...]
    acc = jnp.dot(a_ud_ref[...], y[:, :out_dim],
                  preferred_element_type=jnp.float32)
    acc = acc + jnp.dot(a_lr_ref[...], y[:, out_dim:],
                        preferred_element_type=jnp.float32)
    bm = out_ref.shape[0]
    acc = acc + base_ref[pl.ds(i * bm, bm), :]
    out_ref[...] = jnp.maximum(acc, 0.0)


def kernel(x, mask, a_ud, a_lr, W_lin, b_lin, W_ud, b_ud, W_lr, b_lr):
    num_sent, sent_len, hidden = x.shape
    n = num_sent * sent_len
    out_dim = W_lin.shape[0]
    xf = x.reshape(n, hidden)
    blin = b_lin.reshape(1, out_dim)
    bud = b_ud.reshape(1, out_dim)
    blr = b_lr.reshape(1, out_dim)

    bm = 512
    grid = (n // bm,)
    h = pl.pallas_call(
        functools.partial(_gnn_block, out_dim),
        grid=grid,
        in_specs=[
            pl.BlockSpec((bm, n), lambda i: (i, 0)),
            pl.BlockSpec((bm, n), lambda i: (i, 0)),
            pl.BlockSpec((n, hidden), lambda i: (0, 0)),
            pl.BlockSpec((out_dim, hidden), lambda i: (0, 0)),
            pl.BlockSpec((out_dim, hidden), lambda i: (0, 0)),
            pl.BlockSpec((out_dim, hidden), lambda i: (0, 0)),
            pl.BlockSpec((1, out_dim), lambda i: (0, 0)),
            pl.BlockSpec((1, out_dim), lambda i: (0, 0)),
            pl.BlockSpec((1, out_dim), lambda i: (0, 0)),
        ],
        out_specs=pl.BlockSpec((bm, out_dim), lambda i: (i, 0)),
        out_shape=jax.ShapeDtypeStruct((n, out_dim), jnp.float32),
        scratch_shapes=[
            pltpu.VMEM((n, 2 * out_dim), jnp.float32),
            pltpu.VMEM((n, out_dim), jnp.float32),
        ],
    )(a_ud, a_lr, xf, W_lin, W_ud, W_lr, blin, bud, blr)
    return h.reshape(num_sent, sent_len, out_dim)


# speedup vs baseline: 1.0196x; 1.0196x over previous
import functools

import jax
import jax.numpy as jnp
from jax.experimental import pallas as pl
from jax.experimental.pallas import tpu as pltpu


def _xwt(xf, w):
    return jax.lax.dot_general(xf, w, (((1,), (1,)), ((), ())),
                               preferred_element_type=jnp.float32)


def _gnn_block(out_dim, a_ud_ref, a_lr_ref, xf_ref, wlin_ref, wud_ref,
               wlr_ref, blin_ref, bud_ref, blr_ref, out_ref, y_ref, base_ref):
    i = pl.program_id(0)

    @pl.when(i == 0)
    def _():
        xf = xf_ref[...]
        y_ref[:, :out_dim] = _xwt(xf, wud_ref[...])
        y_ref[:, out_dim:] = _xwt(xf, wlr_ref[...])
        base_ref[...] = (_xwt(xf, wlin_ref[...])
                         + (blin_ref[...] + bud_ref[...] + blr_ref[...]))

    y = y_ref[...]
    acc = jnp.dot(a_ud_ref[...], y[:, :out_dim],
                  preferred_element_type=jnp.float32)
    acc = acc + jnp.dot(a_lr_ref[...], y[:, out_dim:],
                        preferred_element_type=jnp.float32)
    bm = out_ref.shape[0]
    acc = acc + base_ref[pl.ds(i * bm, bm), :]
    out_ref[...] = jnp.maximum(acc, 0.0)


def kernel(x, mask, a_ud, a_lr, W_lin, b_lin, W_ud, b_ud, W_lr, b_lr):
    num_sent, sent_len, hidden = x.shape
    n = num_sent * sent_len
    out_dim = W_lin.shape[0]
    xf = x.reshape(n, hidden)
    blin = b_lin.reshape(1, out_dim)
    bud = b_ud.reshape(1, out_dim)
    blr = b_lr.reshape(1, out_dim)

    bm = 256
    grid = (n // bm,)
    h = pl.pallas_call(
        functools.partial(_gnn_block, out_dim),
        grid=grid,
        in_specs=[
            pl.BlockSpec((bm, n), lambda i: (i, 0)),
            pl.BlockSpec((bm, n), lambda i: (i, 0)),
            pl.BlockSpec((n, hidden), lambda i: (0, 0)),
            pl.BlockSpec((out_dim, hidden), lambda i: (0, 0)),
            pl.BlockSpec((out_dim, hidden), lambda i: (0, 0)),
            pl.BlockSpec((out_dim, hidden), lambda i: (0, 0)),
            pl.BlockSpec((1, out_dim), lambda i: (0, 0)),
            pl.BlockSpec((1, out_dim), lambda i: (0, 0)),
            pl.BlockSpec((1, out_dim), lambda i: (0, 0)),
        ],
        out_specs=pl.BlockSpec((bm, out_dim), lambda i: (i, 0)),
        out_shape=jax.ShapeDtypeStruct((n, out_dim), jnp.float32),
        scratch_shapes=[
            pltpu.VMEM((n, 2 * out_dim), jnp.float32),
            pltpu.VMEM((n, out_dim), jnp.float32),
        ],
    )(a_ud, a_lr, xf, W_lin, W_ud, W_lr, blin, bud, blr)
    return h.reshape(num_sent, sent_len, out_dim)
